# baseline (speedup 1.0000x reference)
import functools

import jax
import jax.numpy as jnp
from jax.experimental import pallas as pl


_BT = 512


def _gate_body(z_ref, w_ref, b_ref, o_ref):
    z = z_ref[...]
    w = w_ref[...]
    logits = jax.lax.dot_general(
        z, w, (((1,), (0,)), ((), ())), preferred_element_type=jnp.float32
    ) + b_ref[...]
    m = jnp.max(logits, axis=-1, keepdims=True)
    e = jnp.exp(logits - m)
    o_ref[...] = e / jnp.sum(e, axis=-1, keepdims=True)


@jax.jit
def kernel(z, W, b):
    n_tokens, d_model = z.shape
    n_exp = W.shape[1]
    grid = n_tokens // _BT
    return pl.pallas_call(
        _gate_body,
        grid=(grid,),
        in_specs=[
            pl.BlockSpec((_BT, d_model), lambda i: (i, 0)),
            pl.BlockSpec((d_model, n_exp), lambda i: (0, 0)),
            pl.BlockSpec((1, n_exp), lambda i: (0, 0)),
        ],
        out_specs=pl.BlockSpec((_BT, n_exp), lambda i: (i, 0)),
        out_shape=jax.ShapeDtypeStruct((n_tokens, n_exp), jnp.float32),
    )(z, W, b.reshape(1, n_exp))


# speedup vs baseline: 1.6779x; 1.6779x over previous
import functools

import jax
import jax.numpy as jnp
from jax.experimental import pallas as pl
from jax.experimental.pallas import tpu as pltpu


_BT = 4096


def _gate_body(z_ref, w_ref, b_ref, o_ref):
    z = z_ref[...]
    w = w_ref[...]
    logits = jax.lax.dot_general(
        z, w, (((1,), (0,)), ((), ())), preferred_element_type=jnp.float32
    ) + b_ref[...]
    m = jnp.max(logits, axis=-1, keepdims=True)
    e = jnp.exp(logits - m)
    o_ref[...] = e / jnp.sum(e, axis=-1, keepdims=True)


@jax.jit
def kernel(z, W, b):
    n_tokens, d_model = z.shape
    n_exp = W.shape[1]
    grid = n_tokens // _BT
    return pl.pallas_call(
        _gate_body,
        grid=(grid,),
        in_specs=[
            pl.BlockSpec((_BT, d_model), lambda i: (i, 0)),
            pl.BlockSpec((d_model, n_exp), lambda i: (0, 0)),
            pl.BlockSpec((1, n_exp), lambda i: (0, 0)),
        ],
        out_specs=pl.BlockSpec((_BT, n_exp), lambda i: (i, 0)),
        out_shape=jax.ShapeDtypeStruct((n_tokens, n_exp), jnp.float32),
        compiler_params=pltpu.CompilerParams(
            dimension_semantics=("arbitrary",),
        ),
    )(z, W, b.reshape(1, n_exp))
